# baseline (speedup 1.0000x reference)
import functools

import jax
import jax.numpy as jnp
from jax import lax
from jax.experimental import pallas as pl
from jax.experimental.pallas import tpu as pltpu
from jax.experimental.pallas import tpu_sc as plsc

B, C, H, W = 2, 96, 512, 512
HW = H * W
NC, NS = 2, 16
NW = NC * NS
PPW = B * HW // NW
M = 128
NCHUNK = PPW // M
N2 = NCHUNK // 2


def _warp_sc(table, dispi, dispj):
    mesh = plsc.VectorSubcoreMesh(core_axis_name="c", subcore_axis_name="s")

    @functools.partial(
        pl.kernel,
        mesh=mesh,
        compiler_params=pltpu.CompilerParams(
            needs_layout_passes=False,
            use_tc_tiling_on_sc=False,
        ),
        out_type=jax.ShapeDtypeStruct((B * HW, C), jnp.float32),
        scratch_types=[
            pltpu.VMEM((2, M), jnp.float32),
            pltpu.VMEM((2, M), jnp.float32),
            pltpu.VMEM((2, M + 16), jnp.float32),
            pltpu.VMEM((2, M + 16), jnp.float32),
            pltpu.VMEM((2, M), jnp.int32),
            pltpu.VMEM((2, M), jnp.int32),
            pltpu.VMEM((2, M), jnp.int32),
            pltpu.VMEM((2, M), jnp.int32),
            pltpu.VMEM((2, M, C), jnp.float32),
            pltpu.VMEM((2, M, C), jnp.float32),
            pltpu.VMEM((2, M, C), jnp.float32),
            pltpu.VMEM((2, M, C), jnp.float32),
            pltpu.VMEM((2, M, C), jnp.float32),
            pltpu.SemaphoreType.DMA,
            pltpu.SemaphoreType.DMA,
            pltpu.SemaphoreType.DMA,
            pltpu.SemaphoreType.DMA,
            pltpu.SemaphoreType.DMA,
            pltpu.SemaphoreType.DMA,
        ],
    )
    def k(table_h, di_h, dj_h, out_h,
          di_v, dj_v, wi_v, wj_v,
          i00, i01, i10, i11,
          g00, g01, g10, g11,
          oU, dsem0, dsem1, gsem0, gsem1, ssem0, ssem1):
        wid = lax.axis_index("s") * NC + lax.axis_index("c")
        pix0 = wid * PPW
        b = pix0 // HW
        tab_base = b * HW
        lane = lax.iota(jnp.int32, 16)
        dsem = (dsem0, dsem1)
        gsem = (gsem0, gsem1)
        ssem = (ssem0, ssem1)

        def fire_disp(t, par, sem):
            p0 = pix0 + t * M
            pltpu.async_copy(di_h.at[pl.ds(p0, M)], di_v.at[par], sem)
            pltpu.async_copy(dj_h.at[pl.ds(p0, M)], dj_v.at[par], sem)

        def wait_disp(par, sem):
            pltpu.make_async_copy(
                di_h.at[pl.ds(0, M)], di_v.at[par], sem).wait()
            pltpu.make_async_copy(
                dj_h.at[pl.ds(0, M)], dj_v.at[par], sem).wait()

        def prep(t, par):
            p0 = pix0 + t * M
            for g in range(M // 16):
                pvec = p0 + g * 16 + lane
                ii = lax.shift_right_logical(pvec, 9) & (H - 1)
                jj = pvec & (W - 1)
                li = ii.astype(jnp.float32) + di_v[par, pl.ds(g * 16, 16)]
                lj = jj.astype(jnp.float32) + dj_v[par, pl.ds(g * 16, 16)]
                lic = jnp.clip(li, 0.0, float(H - 1))
                ljc = jnp.clip(lj, 0.0, float(W - 1))
                ai = jnp.minimum(lic.astype(jnp.int32), H - 2)
                aj = jnp.minimum(ljc.astype(jnp.int32), W - 2)
                wi_v[par, pl.ds(g * 16, 16)] = lic - ai.astype(jnp.float32)
                wj_v[par, pl.ds(g * 16, 16)] = ljc - aj.astype(jnp.float32)
                r00 = tab_base + ai * W + aj
                i00[par, pl.ds(g * 16, 16)] = r00
                i01[par, pl.ds(g * 16, 16)] = r00 + 1
                i10[par, pl.ds(g * 16, 16)] = r00 + W
                i11[par, pl.ds(g * 16, 16)] = r00 + W + 1
            sem = gsem[par]
            pltpu.async_copy(table_h.at[i00.at[par]], g00.at[par], sem)
            pltpu.async_copy(table_h.at[i01.at[par]], g01.at[par], sem)
            pltpu.async_copy(table_h.at[i10.at[par]], g10.at[par], sem)
            pltpu.async_copy(table_h.at[i11.at[par]], g11.at[par], sem)

        def wait_gathers(par):
            sem = gsem[par]
            for gbuf, ibuf in ((g00, i00), (g01, i01), (g10, i10), (g11, i11)):
                pltpu.make_async_copy(
                    table_h.at[ibuf.at[par]], gbuf.at[par], sem).wait()

        def blend(par):
            def pix(p, cc):
                w_i = jnp.full((16,), wi_v[par, pl.ds(p, 16)][0], jnp.float32)
                w_j = jnp.full((16,), wj_v[par, pl.ds(p, 16)][0], jnp.float32)
                for q in range(C // 16):
                    sl = pl.ds(q * 16, 16)
                    v00 = g00[par, p, sl]
                    v01 = g01[par, p, sl]
                    v10 = g10[par, p, sl]
                    v11 = g11[par, p, sl]
                    top = v00 + w_j * (v01 - v00)
                    bot = v10 + w_j * (v11 - v10)
                    oU[par, p, sl] = top + w_i * (bot - top)
                return cc
            lax.fori_loop(0, M, pix, 0, unroll=4)

        def fire_out(t, par):
            p0 = pix0 + t * M
            pltpu.async_copy(oU.at[par], out_h.at[pl.ds(p0, M)], ssem[par])

        def wait_out(par):
            pltpu.make_async_copy(
                oU.at[par], out_h.at[pl.ds(0, M)], ssem[par]).wait()

        fire_disp(0, 0, dsem0)
        fire_disp(1, 1, dsem1)
        wait_disp(0, dsem0)
        prep(0, 0)

        def body(u, carry):
            t0 = 2 * u
            t1 = t0 + 1
            @pl.when(u < N2 - 1)
            def _():
                fire_disp(t0 + 2, 0, dsem0)
            wait_disp(1, dsem1)
            prep(t1, 1)
            wait_gathers(0)

            @pl.when(u > 0)
            def _():
                wait_out(0)
            blend(0)
            fire_out(t0, 0)
            @pl.when(u < N2 - 1)
            def _():
                fire_disp(t1 + 2, 1, dsem1)
                wait_disp(0, dsem0)
                prep(t0 + 2, 0)
            wait_gathers(1)

            @pl.when(u > 0)
            def _():
                wait_out(1)
            blend(1)
            fire_out(t1, 1)
            return carry

        lax.fori_loop(0, N2, body, 0)
        wait_out(0)
        wait_out(1)

    return k(table, dispi, dispj)


def kernel(img, trf):
    table = jnp.transpose(img, (0, 2, 3, 1)).reshape(B * HW, C)
    dispi = trf[:, 0].reshape(B * HW)
    dispj = trf[:, 1].reshape(B * HW)
    out = _warp_sc(table, dispi, dispj)
    return jnp.transpose(out.reshape(B, H, W, C), (0, 3, 1, 2))


# speedup vs baseline: 1.4308x; 1.4308x over previous
import functools

import jax
import jax.numpy as jnp
from jax import lax
from jax.experimental import pallas as pl
from jax.experimental.pallas import tpu as pltpu
from jax.experimental.pallas import tpu_sc as plsc

B, C, H, W = 2, 96, 512, 512
CPAD = 128
HW = H * W
NC, NS = 2, 16
NW = NC * NS
PPW = B * HW // NW
M = 128
NCHUNK = PPW // M


def _warp_sc(table, dispi, dispj):
    mesh = plsc.VectorSubcoreMesh(core_axis_name="c", subcore_axis_name="s")

    @functools.partial(
        pl.kernel,
        mesh=mesh,
        compiler_params=pltpu.CompilerParams(
            needs_layout_passes=False,
        ),
        out_type=jax.ShapeDtypeStruct((B * HW, C), jnp.float32),
        scratch_types=[
            pltpu.VMEM((M,), jnp.float32),
            pltpu.VMEM((M,), jnp.float32),
            pltpu.VMEM((M + 16,), jnp.float32),
            pltpu.VMEM((M + 16,), jnp.float32),
            pltpu.VMEM((M,), jnp.int32),
            pltpu.VMEM((M,), jnp.int32),
            pltpu.VMEM((M,), jnp.int32),
            pltpu.VMEM((M,), jnp.int32),
            pltpu.VMEM((M, CPAD), jnp.float32),
            pltpu.VMEM((M, CPAD), jnp.float32),
            pltpu.VMEM((M, CPAD), jnp.float32),
            pltpu.VMEM((M, CPAD), jnp.float32),
            pltpu.VMEM((M, C), jnp.float32),
            pltpu.SemaphoreType.DMA,
            pltpu.SemaphoreType.DMA,
        ],
    )
    def k(table_h, di_h, dj_h, out_h,
          di_v, dj_v, wi_v, wj_v,
          i00, i01, i10, i11,
          g00, g01, g10, g11,
          oU, gsem, ssem):
        wid = lax.axis_index("s") * NC + lax.axis_index("c")
        pix0 = wid * PPW
        b = pix0 // HW
        tab_base = b * HW
        lane = lax.iota(jnp.int32, 16)

        def chunk(t, carry):
            p0 = pix0 + t * M
            pltpu.sync_copy(di_h.at[pl.ds(p0, M)], di_v)
            pltpu.sync_copy(dj_h.at[pl.ds(p0, M)], dj_v)

            for g in range(M // 16):
                pvec = p0 + g * 16 + lane
                ii = lax.shift_right_logical(pvec, 9) & (H - 1)
                jj = pvec & (W - 1)
                li = ii.astype(jnp.float32) + di_v[pl.ds(g * 16, 16)]
                lj = jj.astype(jnp.float32) + dj_v[pl.ds(g * 16, 16)]
                lic = jnp.clip(li, 0.0, float(H - 1))
                ljc = jnp.clip(lj, 0.0, float(W - 1))
                ai = jnp.minimum(lic.astype(jnp.int32), H - 2)
                aj = jnp.minimum(ljc.astype(jnp.int32), W - 2)
                wi_v[pl.ds(g * 16, 16)] = lic - ai.astype(jnp.float32)
                wj_v[pl.ds(g * 16, 16)] = ljc - aj.astype(jnp.float32)
                r00 = tab_base + ai * W + aj
                i00[pl.ds(g * 16, 16)] = r00
                i01[pl.ds(g * 16, 16)] = r00 + 1
                i10[pl.ds(g * 16, 16)] = r00 + W
                i11[pl.ds(g * 16, 16)] = r00 + W + 1

            c0 = pltpu.async_copy(table_h.at[i00], g00, gsem)
            c1 = pltpu.async_copy(table_h.at[i01], g01, gsem)
            c2 = pltpu.async_copy(table_h.at[i10], g10, gsem)
            c3 = pltpu.async_copy(table_h.at[i11], g11, gsem)
            c0.wait()
            c1.wait()
            c2.wait()
            c3.wait()

            def pix(p, cc):
                w_i = jnp.full((16,), wi_v[pl.ds(p, 16)][0], jnp.float32)
                w_j = jnp.full((16,), wj_v[pl.ds(p, 16)][0], jnp.float32)
                for q in range(C // 16):
                    sl = pl.ds(q * 16, 16)
                    v00 = g00[p, sl]
                    v01 = g01[p, sl]
                    v10 = g10[p, sl]
                    v11 = g11[p, sl]
                    top = v00 + w_j * (v01 - v00)
                    bot = v10 + w_j * (v11 - v10)
                    oU[p, sl] = top + w_i * (bot - top)
                return cc

            lax.fori_loop(0, M, pix, 0)

            pltpu.async_copy(oU, out_h.at[pl.ds(p0, M)], ssem).wait()
            return carry

        lax.fori_loop(0, NCHUNK, chunk, 0)

    return k(table, dispi, dispj)


def kernel(img, trf):
    table = jnp.pad(
        jnp.transpose(img, (0, 2, 3, 1)).reshape(B * HW, C),
        ((0, 0), (0, CPAD - C)))
    dispi = trf[:, 0].reshape(B * HW)
    dispj = trf[:, 1].reshape(B * HW)
    out = _warp_sc(table, dispi, dispj)
    return jnp.transpose(out.reshape(B, H, W, C), (0, 3, 1, 2))


# speedup vs baseline: 2.0426x; 1.4276x over previous
import functools

import jax
import jax.numpy as jnp
from jax import lax
from jax.experimental import pallas as pl
from jax.experimental.pallas import tpu as pltpu
from jax.experimental.pallas import tpu_sc as plsc

B, C, H, W = 2, 96, 512, 512
CPAD = 128
HW = H * W
NC, NS = 2, 16
NW = NC * NS
PPW = B * HW // NW
M = 64
NCHUNK = PPW // M
N2 = NCHUNK // 2


def _warp_sc(table, dispi, dispj):
    mesh = plsc.VectorSubcoreMesh(core_axis_name="c", subcore_axis_name="s")

    @functools.partial(
        pl.kernel,
        mesh=mesh,
        compiler_params=pltpu.CompilerParams(
            needs_layout_passes=False,
        ),
        out_type=jax.ShapeDtypeStruct((B * HW, C), jnp.float32),
        scratch_types=[
            pltpu.VMEM((2, M), jnp.float32),
            pltpu.VMEM((2, M), jnp.float32),
            pltpu.VMEM((2, M + 16), jnp.float32),
            pltpu.VMEM((2, M + 16), jnp.float32),
            pltpu.VMEM((2, M), jnp.int32),
            pltpu.VMEM((2, M), jnp.int32),
            pltpu.VMEM((2, M), jnp.int32),
            pltpu.VMEM((2, M), jnp.int32),
            pltpu.VMEM((2, M, CPAD), jnp.float32),
            pltpu.VMEM((2, M, CPAD), jnp.float32),
            pltpu.VMEM((2, M, CPAD), jnp.float32),
            pltpu.VMEM((2, M, CPAD), jnp.float32),
            pltpu.VMEM((2, M, C), jnp.float32),
            pltpu.SemaphoreType.DMA,
            pltpu.SemaphoreType.DMA,
            pltpu.SemaphoreType.DMA,
            pltpu.SemaphoreType.DMA,
            pltpu.SemaphoreType.DMA,
            pltpu.SemaphoreType.DMA,
        ],
    )
    def k(table_h, di_h, dj_h, out_h,
          di_v, dj_v, wi_v, wj_v,
          i00, i01, i10, i11,
          g00, g01, g10, g11,
          oU, dsem0, dsem1, gsem0, gsem1, ssem0, ssem1):
        wid = lax.axis_index("s") * NC + lax.axis_index("c")
        pix0 = wid * PPW
        b = pix0 // HW
        tab_base = b * HW
        lane = lax.iota(jnp.int32, 16)
        dsem = (dsem0, dsem1)
        gsem = (gsem0, gsem1)
        ssem = (ssem0, ssem1)

        def fire_disp(t, par):
            p0 = pix0 + t * M
            pltpu.async_copy(di_h.at[pl.ds(p0, M)], di_v.at[par], dsem[par])
            pltpu.async_copy(dj_h.at[pl.ds(p0, M)], dj_v.at[par], dsem[par])

        def wait_disp(par):
            pltpu.make_async_copy(
                di_h.at[pl.ds(0, M)], di_v.at[par], dsem[par]).wait()
            pltpu.make_async_copy(
                dj_h.at[pl.ds(0, M)], dj_v.at[par], dsem[par]).wait()

        def prep(t, par):
            p0 = pix0 + t * M
            for g in range(M // 16):
                pvec = p0 + g * 16 + lane
                ii = lax.shift_right_logical(pvec, 9) & (H - 1)
                jj = pvec & (W - 1)
                li = ii.astype(jnp.float32) + di_v[par, pl.ds(g * 16, 16)]
                lj = jj.astype(jnp.float32) + dj_v[par, pl.ds(g * 16, 16)]
                lic = jnp.clip(li, 0.0, float(H - 1))
                ljc = jnp.clip(lj, 0.0, float(W - 1))
                ai = jnp.minimum(lic.astype(jnp.int32), H - 2)
                aj = jnp.minimum(ljc.astype(jnp.int32), W - 2)
                wi_v[par, pl.ds(g * 16, 16)] = lic - ai.astype(jnp.float32)
                wj_v[par, pl.ds(g * 16, 16)] = ljc - aj.astype(jnp.float32)
                r00 = tab_base + ai * W + aj
                i00[par, pl.ds(g * 16, 16)] = r00
                i01[par, pl.ds(g * 16, 16)] = r00 + 1
                i10[par, pl.ds(g * 16, 16)] = r00 + W
                i11[par, pl.ds(g * 16, 16)] = r00 + W + 1
            sem = gsem[par]
            pltpu.async_copy(table_h.at[i00.at[par]], g00.at[par], sem)
            pltpu.async_copy(table_h.at[i01.at[par]], g01.at[par], sem)
            pltpu.async_copy(table_h.at[i10.at[par]], g10.at[par], sem)
            pltpu.async_copy(table_h.at[i11.at[par]], g11.at[par], sem)

        def wait_gathers(par):
            sem = gsem[par]
            for gbuf, ibuf in ((g00, i00), (g01, i01), (g10, i10), (g11, i11)):
                pltpu.make_async_copy(
                    table_h.at[ibuf.at[par]], gbuf.at[par], sem).wait()

        def blend(par):
            def pix(p, cc):
                w_i = jnp.full((16,), wi_v[par, pl.ds(p, 16)][0], jnp.float32)
                w_j = jnp.full((16,), wj_v[par, pl.ds(p, 16)][0], jnp.float32)
                for q in range(C // 16):
                    sl = pl.ds(q * 16, 16)
                    v00 = g00[par, p, sl]
                    v01 = g01[par, p, sl]
                    v10 = g10[par, p, sl]
                    v11 = g11[par, p, sl]
                    top = v00 + w_j * (v01 - v00)
                    bot = v10 + w_j * (v11 - v10)
                    oU[par, p, sl] = top + w_i * (bot - top)
                return cc
            lax.fori_loop(0, M, pix, 0)

        def fire_out(t, par):
            p0 = pix0 + t * M
            pltpu.async_copy(oU.at[par], out_h.at[pl.ds(p0, M)], ssem[par])

        def wait_out(par):
            pltpu.make_async_copy(
                oU.at[par], out_h.at[pl.ds(0, M)], ssem[par]).wait()

        fire_disp(0, 0)
        fire_disp(1, 1)
        wait_disp(0)
        prep(0, 0)

        def body(u, carry):
            t0 = 2 * u
            t1 = t0 + 1
            @pl.when(u < N2 - 1)
            def _():
                fire_disp(t0 + 2, 0)
            wait_disp(1)
            prep(t1, 1)
            wait_gathers(0)

            @pl.when(u > 0)
            def _():
                wait_out(0)
            blend(0)
            fire_out(t0, 0)
            @pl.when(u < N2 - 1)
            def _():
                fire_disp(t1 + 2, 1)
                wait_disp(0)
                prep(t0 + 2, 0)
            wait_gathers(1)

            @pl.when(u > 0)
            def _():
                wait_out(1)
            blend(1)
            fire_out(t1, 1)
            return carry

        lax.fori_loop(0, N2, body, 0)
        wait_out(0)
        wait_out(1)

    return k(table, dispi, dispj)


def kernel(img, trf):
    table = jnp.pad(
        jnp.transpose(img, (0, 2, 3, 1)).reshape(B * HW, C),
        ((0, 0), (0, CPAD - C)))
    dispi = trf[:, 0].reshape(B * HW)
    dispj = trf[:, 1].reshape(B * HW)
    out = _warp_sc(table, dispi, dispj)
    return jnp.transpose(out.reshape(B, H, W, C), (0, 3, 1, 2))
